# baseline (speedup 1.0000x reference)
import jax
import jax.numpy as jnp
from jax import lax
from jax.experimental import pallas as pl
from jax.experimental.pallas import tpu as pltpu


def _mask_mul_body(sites_ref, x_ref, o_ref, mask_ref):
    @pl.when(pl.program_id(0) == 0)
    def _():
        s_pad = sites_ref.shape[0]
        w = x_ref.shape[-1]
        sites = sites_ref[...]
        cols = lax.broadcasted_iota(jnp.int32, (s_pad, w), 1)
        hit = jnp.any(sites == cols, axis=0, keepdims=True)
        mask_ref[...] = jnp.where(hit, 0.0, 1.0)

    o_ref[...] = x_ref[...] * mask_ref[...][None]


def _mul_part(sites, xpart, bc):
    c, h, w = xpart.shape
    s_pad = sites.shape[0]
    return pl.pallas_call(
        _mask_mul_body,
        grid=(c // bc,),
        in_specs=[
            pl.BlockSpec((s_pad, 1), lambda i: (0, 0)),
            pl.BlockSpec((bc, h, w), lambda i: (i, 0, 0)),
        ],
        out_specs=pl.BlockSpec((bc, h, w), lambda i: (i, 0, 0)),
        out_shape=jax.ShapeDtypeStruct((c, h, w), xpart.dtype),
        scratch_shapes=[pltpu.VMEM((1, w), jnp.float32)],
    )(sites, xpart)


def kernel(x, mask_sites):
    c, h, w = x.shape
    n = mask_sites.shape[0]
    s_pad = (n + 7) // 8 * 8
    sites = jnp.full((s_pad, 1), w, dtype=jnp.int32)
    sites = lax.dynamic_update_slice(
        sites, mask_sites.astype(jnp.int32).reshape(n, 1), (0, 0)
    )
    half = c // 2
    a = _mul_part(sites, x[:half], 12)
    b = _mul_part(sites, x[half:], 12)
    return jnp.concatenate([a, b], axis=0)


# speedup vs baseline: 2.6428x; 2.6428x over previous
import functools

import jax
import jax.numpy as jnp
from jax import lax
from jax.experimental import pallas as pl
from jax.experimental.pallas import tpu as pltpu
from jax.experimental.pallas import tpu_sc as plsc

_LANES = 16


def _sc_mask_body(w, s_pad, sites_hbm, mask_hbm, sites_v, mask_v):
    wid = lax.axis_index("s") * 2 + lax.axis_index("c")

    @pl.when(wid == 0)
    def _():
        pltpu.sync_copy(sites_hbm, sites_v)
        for i in range(w // _LANES):
            mask_v[pl.ds(i * _LANES, _LANES)] = jnp.ones((_LANES,), jnp.float32)
        for i in range(s_pad // _LANES):
            idx = sites_v[pl.ds(i * _LANES, _LANES)]
            plsc.store_scatter(mask_v, [idx], jnp.zeros((_LANES,), jnp.float32))
        pltpu.sync_copy(mask_v, mask_hbm.at[0])


def _tc_mul_body(mask_ref, x_ref, o_ref):
    o_ref[...] = x_ref[...] * mask_ref[...][None]


def kernel(x, mask_sites):
    c, h, w = x.shape
    n = mask_sites.shape[0]
    s_pad = (n + _LANES - 1) // _LANES * _LANES
    sites = mask_sites.astype(jnp.int32)
    sites = jnp.concatenate([sites, jnp.broadcast_to(sites[:1], (s_pad - n,))])

    sc_mask = functools.partial(
        pl.kernel,
        mesh=plsc.VectorSubcoreMesh(
            core_axis_name="c", subcore_axis_name="s", num_cores=1
        ),
        compiler_params=pltpu.CompilerParams(needs_layout_passes=False),
        out_type=jax.ShapeDtypeStruct((1, w), jnp.float32),
        scratch_types=[
            pltpu.VMEM((s_pad,), jnp.int32),
            pltpu.VMEM((w,), jnp.float32),
        ],
    )(functools.partial(_sc_mask_body, w, s_pad))
    mask = sc_mask(sites)

    bc = 12
    return pl.pallas_call(
        _tc_mul_body,
        grid=(c // bc,),
        in_specs=[
            pl.BlockSpec((1, w), lambda i: (0, 0)),
            pl.BlockSpec((bc, h, w), lambda i: (i, 0, 0)),
        ],
        out_specs=pl.BlockSpec((bc, h, w), lambda i: (i, 0, 0)),
        out_shape=jax.ShapeDtypeStruct((c, h, w), x.dtype),
    )(mask, x)


# speedup vs baseline: 3.0062x; 1.1375x over previous
import jax
import jax.numpy as jnp
from jax import lax
from jax.experimental import pallas as pl
from jax.experimental.pallas import tpu as pltpu


def _mask_mul_body(sites_ref, x_ref, o_ref, mask_ref):
    @pl.when(pl.program_id(0) == 0)
    def _():
        s_pad = sites_ref.shape[0]
        w = x_ref.shape[-1]
        sites = sites_ref[...]
        cols = lax.broadcasted_iota(jnp.int32, (s_pad, w), 1)
        hit = jnp.any(sites == cols, axis=0, keepdims=True)
        mask_ref[...] = jnp.where(hit, 0.0, 1.0)

    o_ref[...] = x_ref[...] * mask_ref[...][None]


def kernel(x, mask_sites):
    c, h, w = x.shape
    n = mask_sites.shape[0]
    s_pad = (n + 7) // 8 * 8
    sites = jnp.full((s_pad, 1), w, dtype=jnp.int32)
    sites = lax.dynamic_update_slice(
        sites, mask_sites.astype(jnp.int32).reshape(n, 1), (0, 0)
    )

    bc = 12
    return pl.pallas_call(
        _mask_mul_body,
        grid=(c // bc,),
        in_specs=[
            pl.BlockSpec((s_pad, 1), lambda i: (0, 0)),
            pl.BlockSpec((bc, h, w), lambda i: (i, 0, 0)),
        ],
        out_specs=pl.BlockSpec((bc, h, w), lambda i: (i, 0, 0)),
        out_shape=jax.ShapeDtypeStruct((c, h, w), x.dtype),
        scratch_shapes=[pltpu.VMEM((1, w), jnp.float32)],
    )(sites, x)


# speedup vs baseline: 3.0102x; 1.0013x over previous
import jax
import jax.numpy as jnp
from jax import lax
from jax.experimental import pallas as pl
from jax.experimental.pallas import tpu as pltpu


def _mask_mul_body(sites_ref, x_ref, o_ref, mask_ref):
    @pl.when(pl.program_id(0) == 0)
    def _():
        s_pad = sites_ref.shape[0]
        w = x_ref.shape[-1]
        sites = sites_ref[...]
        cols = lax.broadcasted_iota(jnp.int32, (s_pad, w), 1)
        hit = jnp.any(sites == cols, axis=0, keepdims=True)
        mask_ref[...] = jnp.where(hit, 0.0, 1.0)

    o_ref[...] = x_ref[...] * mask_ref[...][None]


def kernel(x, mask_sites):
    c, h, w = x.shape
    n = mask_sites.shape[0]
    sites = mask_sites.astype(jnp.int32).reshape(n, 1)

    bc = 12
    return pl.pallas_call(
        _mask_mul_body,
        grid=(c // bc,),
        in_specs=[
            pl.BlockSpec((n, 1), lambda i: (0, 0)),
            pl.BlockSpec((bc, h, w), lambda i: (i, 0, 0)),
        ],
        out_specs=pl.BlockSpec((bc, h, w), lambda i: (i, 0, 0)),
        out_shape=jax.ShapeDtypeStruct((c, h, w), x.dtype),
        scratch_shapes=[pltpu.VMEM((1, w), jnp.float32)],
    )(sites, x)
